# Initial kernel scaffold; baseline (speedup 1.0000x reference)
import functools

import jax
import jax.numpy as jnp
from jax import lax
from jax.experimental import pallas as pl
from jax.experimental.pallas import tpu as pltpu
from jax.experimental.pallas import tpu_sc as plsc

N = 10000
E = 320000
D = 128
H = 4
DH = 32
PROJ = 128
B = 64

NC = 2
NS = 16
NW = NC * NS
EPT = E // NW
KE = 80
NBLK = EPT // KE
ROWW = 144
RPT = N // NS
NROW = 1000
NG = N // NROW


def _dense_body(x_ref, w_ref, a2_ref, h_ref, ee_ref):
    h = jnp.dot(x_ref[...], w_ref[...], preferred_element_type=jnp.float32)
    h_ref[...] = h
    ee_ref[...] = jnp.dot(h, a2_ref[...], preferred_element_type=jnp.float32)


def _dense(x, W, A2):
    return pl.pallas_call(
        _dense_body,
        grid=(NG,),
        in_specs=[
            pl.BlockSpec((NROW, D), lambda i: (i, 0)),
            pl.BlockSpec((D, H * DH), lambda i: (0, 0)),
            pl.BlockSpec((D, 2 * H), lambda i: (0, 0)),
        ],
        out_specs=[
            pl.BlockSpec((NROW, H * DH), lambda i: (i, 0)),
            pl.BlockSpec((NROW, 2 * H), lambda i: (i, 0)),
        ],
        out_shape=[
            jax.ShapeDtypeStruct((N, H * DH), jnp.float32),
            jax.ShapeDtypeStruct((N, 2 * H), jnp.float32),
        ],
    )(x, W, A2)


def _edge_body(src_hbm, dst_hbm, ee_hbm, h_hbm, acc_hbm,
               ee_v, src_v, dst_v, rows_v, ext_v, wsp_v, acc_sh, gsem):
    c = lax.axis_index("c")
    s = lax.axis_index("s")
    wid = s * NC + c

    zeros16 = jnp.zeros((16,), jnp.float32)

    def _zw(i, carry):
        wsp_v[pl.ds(i * 16, 16)] = zeros16
        return carry
    lax.fori_loop(0, (16 * KE) // 16, _zw, 0)

    def _ze(e, carry):
        for q in range(ROWW // 16):
            ext_v[e, pl.ds(q * 16, 16)] = zeros16
        return carry
    lax.fori_loop(0, KE, _ze, 0)

    base = s * RPT
    for t in range(RPT // KE):
        pltpu.sync_copy(ext_v, acc_sh.at[pl.ds(base + t * KE, KE)])
    rem = RPT - (RPT // KE) * KE
    if rem:
        pltpu.sync_copy(ext_v.at[pl.ds(0, rem)],
                        acc_sh.at[pl.ds(base + (RPT // KE) * KE, rem)])
    plsc.subcore_barrier()

    pltpu.sync_copy(src_hbm.at[wid], src_v)
    pltpu.sync_copy(dst_hbm.at[wid], dst_v)
    pltpu.sync_copy(ee_hbm, ee_v)

    def _block(j, carry):
        pltpu.async_copy(h_hbm.at[src_v.at[j]], rows_v, gsem).wait()

        def _wgrp(g, carry2):
            sv = src_v[j, pl.ds(g * 16, 16)] * 8
            dv = dst_v[j, pl.ds(g * 16, 16)] * 8
            for head in range(H):
                se = plsc.load_gather(ee_v, [sv + head])
                de = plsc.load_gather(ee_v, [dv + (H + head)])
                l = se + de
                l = jnp.where(l >= 0.0, l, l * 0.2)
                wsp_v[pl.ds(head * KE + g * 16, 16)] = jnp.exp(l)
            return carry2
        lax.fori_loop(0, KE // 16, _wgrp, 0)

        ilane = lax.iota(jnp.int32, 16)

        def _edge(e, carry2):
            tail = plsc.load_gather(wsp_v, [ilane * KE + e])
            ext_v[e, pl.ds(128, 16)] = tail
            for head in range(H):
                wspl = plsc.load_gather(
                    wsp_v, [jnp.broadcast_to(head * KE + e, (16,))])
                for q in range(2):
                    col = head * DH + q * 16
                    ext_v[e, pl.ds(col, 16)] = rows_v[e, pl.ds(col, 16)] * wspl
            return carry2
        lax.fori_loop(0, KE, _edge, 0)

        pltpu.sync_copy(ext_v, acc_sh.at[dst_v.at[j]], add=True)
        return carry
    lax.fori_loop(0, NBLK, _block, 0)

    plsc.subcore_barrier()
    pltpu.sync_copy(acc_sh.at[pl.ds(base, RPT)],
                    acc_hbm.at[c].at[pl.ds(base, RPT)])


def _edge(src, dst, ee_flat, h):
    mesh = plsc.VectorSubcoreMesh(core_axis_name="c", subcore_axis_name="s",
                                  num_cores=NC, num_subcores=NS)
    fn = pl.kernel(
        _edge_body,
        out_type=jax.ShapeDtypeStruct((NC, N, ROWW), jnp.float32),
        mesh=mesh,
        scratch_types=[
            pltpu.VMEM((N * 2 * H,), jnp.float32),
            pltpu.VMEM((NBLK, KE), jnp.int32),
            pltpu.VMEM((NBLK, KE), jnp.int32),
            pltpu.VMEM((KE, H * DH), jnp.float32),
            pltpu.VMEM((KE, ROWW), jnp.float32),
            pltpu.VMEM((16 * KE,), jnp.float32),
            pltpu.VMEM_SHARED((N, ROWW), jnp.float32),
            pltpu.SemaphoreType.DMA,
        ],
    )
    return fn(src, dst, ee_flat, h)


def _post_body(acc_ref, gf_ref, exp_ref, sums_ref):
    i = pl.program_id(0)
    a = acc_ref[...]
    u = a[0] + a[1]
    msg = u[:, :H * DH]
    den4 = u[:, H * DH:H * DH + H]
    den = jnp.dot(den4, exp_ref[...], preferred_element_type=jnp.float32)
    o = msg / (den + 1e-9)
    o = jnp.where(o > 0.0, o, jnp.exp(jnp.minimum(o, 0.0)) - 1.0)
    gid = gf_ref[...]
    iota = lax.broadcasted_iota(jnp.float32, (1, B), 1)
    oh = (gid == iota).astype(jnp.float32)
    ext = jnp.concatenate([o, jnp.ones((NROW, 1), jnp.float32)], axis=1)
    part = lax.dot_general(oh, ext, (((0,), (0,)), ((), ())),
                           preferred_element_type=jnp.float32)

    @pl.when(i == 0)
    def _():
        sums_ref[...] = part

    @pl.when(i > 0)
    def _():
        sums_ref[...] += part


def _post(acc, gf, Expand):
    return pl.pallas_call(
        _post_body,
        grid=(NG,),
        in_specs=[
            pl.BlockSpec((NC, NROW, ROWW), lambda i: (0, i, 0)),
            pl.BlockSpec((NROW, 1), lambda i: (i, 0)),
            pl.BlockSpec((H, H * DH), lambda i: (0, 0)),
        ],
        out_specs=pl.BlockSpec((B, H * DH + 1), lambda i: (0, 0)),
        out_shape=jax.ShapeDtypeStruct((B, H * DH + 1), jnp.float32),
    )(acc, gf, Expand)


def _final_body(sums_ref, sc_ref, w2_ref, b2_ref, w3_ref, b3_ref, out_ref):
    sums = sums_ref[...]
    cnt = sums[:, H * DH:H * DH + 1]
    pooled = sums[:, :H * DH] / jnp.maximum(cnt, 1.0)
    proj = jnp.dot(pooled, w2_ref[...], preferred_element_type=jnp.float32)
    proj = jnp.maximum(proj + b2_ref[...], 0.0)
    feat = jnp.concatenate([proj, sc_ref[...]], axis=1)
    out_ref[...] = jnp.dot(feat, w3_ref[...],
                           preferred_element_type=jnp.float32) + b3_ref[...]


def _final(sums, scores, W2, b2, W3, b3):
    return pl.pallas_call(
        _final_body,
        out_shape=jax.ShapeDtypeStruct((B, 1), jnp.float32),
    )(sums, scores, W2, b2.reshape(1, PROJ), W3, b3.reshape(1, 1))


def kernel(x, edge_index, graph_ids, scores, W, a_src, a_dst, W2, b2, W3, b3):
    src = edge_index[0].astype(jnp.int32).reshape(NW, NBLK, KE)
    dst = edge_index[1].astype(jnp.int32).reshape(NW, NBLK, KE)

    rows = jnp.arange(D)
    head = rows // DH
    A2 = jnp.zeros((D, 2 * H), jnp.float32)
    A2 = A2.at[rows, head].set(a_src.reshape(-1))
    A2 = A2.at[rows, H + head].set(a_dst.reshape(-1))

    cols = jnp.arange(H * DH)
    Expand = (cols[None, :] // DH == jnp.arange(H)[:, None]).astype(jnp.float32)

    h, ee = _dense(x, W, A2)
    acc = _edge(src, dst, ee.reshape(N * 2 * H), h)
    gf = graph_ids.astype(jnp.float32).reshape(N, 1)
    sums = _post(acc, gf, Expand)
    return _final(sums, scores, W2, b2, W3, b3)


# speedup vs baseline: 54.3986x; 54.3986x over previous
import functools

import jax
import jax.numpy as jnp
from jax import lax
from jax.experimental import pallas as pl
from jax.experimental.pallas import tpu as pltpu
from jax.experimental.pallas import tpu_sc as plsc

N = 10000
E = 320000
D = 128
H = 4
DH = 32
PROJ = 128
B = 64

NC = 2
NS = 16
NW = NC * NS
EPT = E // NW
KE = 80
NBLK = EPT // KE
MW = H * DH
NPAD = 10240
RPT = NPAD // NS
NROW = 1000
NG = N // NROW


def _dense_body(x_ref, w_ref, a2_ref, h_ref, ee_ref):
    h = jnp.dot(x_ref[...], w_ref[...], preferred_element_type=jnp.float32)
    h_ref[...] = h
    ee_ref[...] = jnp.dot(h, a2_ref[...], preferred_element_type=jnp.float32)


def _dense(x, W, A2):
    return pl.pallas_call(
        _dense_body,
        grid=(NG,),
        in_specs=[
            pl.BlockSpec((NROW, D), lambda i: (i, 0)),
            pl.BlockSpec((D, H * DH), lambda i: (0, 0)),
            pl.BlockSpec((D, 2 * H), lambda i: (0, 0)),
        ],
        out_specs=[
            pl.BlockSpec((NROW, H * DH), lambda i: (i, 0)),
            pl.BlockSpec((NROW, 2 * H), lambda i: (i, 0)),
        ],
        out_shape=[
            jax.ShapeDtypeStruct((N, H * DH), jnp.float32),
            jax.ShapeDtypeStruct((N, 2 * H), jnp.float32),
        ],
    )(x, W, A2)


EPP = 10240
SUB = 1280
NSUB = EPP // SUB
SPB = SUB // KE
WBL = 384
NHP = 40064
EEP = 80128


def _wden_body(src_hbm, dst_hbm, ee_hbm, w_hbm, den_hbm,
               ee_v, src_c, dst_c, wbuf_c, den_v):
    c = lax.axis_index("c")
    s = lax.axis_index("s")
    wid = s * NC + c

    zeros16 = jnp.zeros((16,), jnp.float32)

    def _zd(i, carry):
        den_v[pl.ds(i * 16, 16)] = zeros16
        return carry
    lax.fori_loop(0, NHP // 16, _zd, 0)

    pltpu.sync_copy(ee_hbm, ee_v)

    def _chunk(q, carry):
        pltpu.sync_copy(src_hbm.at[wid].at[pl.ds(q * SUB, SUB)], src_c)
        pltpu.sync_copy(dst_hbm.at[wid].at[pl.ds(q * SUB, SUB)], dst_c)

        def _sub(r, carry2):
            def _wgrp(g, carry3):
                off = r * KE + g * 16
                sv = src_c[pl.ds(off, 16)] * (2 * H)
                dvn = dst_c[pl.ds(off, 16)]
                dv = dvn * (2 * H)
                for head in range(H):
                    se = plsc.load_gather(ee_v, [sv + head])
                    de = plsc.load_gather(ee_v, [dv + (H + head)])
                    l = se + de
                    l = jnp.where(l >= 0.0, l, l * 0.2)
                    w = jnp.exp(l)
                    wbuf_c[pl.ds(r * WBL + head * KE + g * 16, 16)] = w
                    plsc.addupdate_scatter(den_v, [dvn * H + head], w)
                return carry3
            return lax.fori_loop(0, KE // 16, _wgrp, carry2)
        lax.fori_loop(0, SPB, _sub, 0)

        pltpu.sync_copy(wbuf_c,
                        w_hbm.at[wid].at[pl.ds(q * SPB * WBL, SPB * WBL)])
        return carry
    lax.fori_loop(0, NSUB, _chunk, 0)

    pltpu.sync_copy(den_v, den_hbm.at[wid])


def _wden(srcP, dstP, ee_flat):
    mesh = plsc.VectorSubcoreMesh(core_axis_name="c", subcore_axis_name="s",
                                  num_cores=NC, num_subcores=NS)
    fn = pl.kernel(
        _wden_body,
        out_type=[
            jax.ShapeDtypeStruct((NW, NSUB * SPB * WBL), jnp.float32),
            jax.ShapeDtypeStruct((NW, NHP), jnp.float32),
        ],
        mesh=mesh,
        scratch_types=[
            pltpu.VMEM((EEP,), jnp.float32),
            pltpu.VMEM((SUB,), jnp.int32),
            pltpu.VMEM((SUB,), jnp.int32),
            pltpu.VMEM((SPB * WBL,), jnp.float32),
            pltpu.VMEM((NHP,), jnp.float32),
        ],
        compiler_params=pltpu.CompilerParams(needs_layout_passes=False),
    )
    return fn(srcP, dstP, ee_flat)


def _scat_body(src_hbm, dst_hbm, w_hbm, h_hbm, acc_hbm,
               src_v, dst_v, rows_v, wv, acc_sh, gsem):
    c = lax.axis_index("c")
    s = lax.axis_index("s")
    wid = s * NC + c

    zeros16 = jnp.zeros((16,), jnp.float32)

    def _ze(e, carry):
        for q in range(MW // 16):
            rows_v[e, pl.ds(q * 16, 16)] = zeros16
        return carry
    lax.fori_loop(0, KE, _ze, 0)

    base = s * RPT
    for t in range(RPT // KE):
        pltpu.sync_copy(rows_v, acc_sh.at[pl.ds(base + t * KE, KE)])
    plsc.subcore_barrier()

    pltpu.sync_copy(src_hbm.at[wid], src_v)
    pltpu.sync_copy(dst_hbm.at[wid], dst_v)

    def _block(j, carry):
        pltpu.async_copy(h_hbm.at[src_v.at[j]], rows_v, gsem).wait()
        pltpu.sync_copy(w_hbm.at[wid].at[pl.ds(j * WBL, WBL)], wv)

        def _edge(e, carry2):
            for head in range(H):
                wspl = plsc.load_gather(
                    wv, [jnp.broadcast_to(head * KE + e, (16,))])
                for q in range(2):
                    col = head * DH + q * 16
                    rows_v[e, pl.ds(col, 16)] = rows_v[e, pl.ds(col, 16)] * wspl
            return carry2
        lax.fori_loop(0, KE, _edge, 0)

        pltpu.sync_copy(rows_v, acc_sh.at[dst_v.at[j]], add=True)
        return carry
    lax.fori_loop(0, NBLK, _block, 0)

    plsc.subcore_barrier()
    pltpu.sync_copy(acc_sh.at[pl.ds(base, RPT)],
                    acc_hbm.at[c].at[pl.ds(base, RPT)])


def _scat(src, dst, w, h):
    mesh = plsc.VectorSubcoreMesh(core_axis_name="c", subcore_axis_name="s",
                                  num_cores=NC, num_subcores=NS)
    fn = pl.kernel(
        _scat_body,
        out_type=jax.ShapeDtypeStruct((NC, NPAD, MW), jnp.float32),
        mesh=mesh,
        scratch_types=[
            pltpu.VMEM((NBLK, KE), jnp.int32),
            pltpu.VMEM((NBLK, KE), jnp.int32),
            pltpu.VMEM((KE, MW), jnp.float32),
            pltpu.VMEM((WBL,), jnp.float32),
            pltpu.VMEM_SHARED((NPAD, MW), jnp.float32),
            pltpu.SemaphoreType.DMA,
        ],
        compiler_params=pltpu.CompilerParams(needs_layout_passes=False),
    )
    return fn(src, dst, w, h)


def _post_body(acc_ref, den_ref, gf_ref, exp_ref, sums_ref):
    i = pl.program_id(0)
    a = acc_ref[...]
    u = a[0] + a[1]
    den4 = jnp.sum(den_ref[...], axis=0)
    den = jnp.dot(den4, exp_ref[...], preferred_element_type=jnp.float32)
    o = u / (den + 1e-9)
    o = jnp.where(o > 0.0, o, jnp.exp(jnp.minimum(o, 0.0)) - 1.0)
    gid = gf_ref[...]
    iota = lax.broadcasted_iota(jnp.int32, (1, B), 1).astype(jnp.float32)
    oh = (gid == iota).astype(jnp.float32)
    ext = jnp.concatenate([o, jnp.ones((NROW, 1), jnp.float32)], axis=1)
    part = lax.dot_general(oh, ext, (((0,), (0,)), ((), ())),
                           preferred_element_type=jnp.float32)

    @pl.when(i == 0)
    def _():
        sums_ref[...] = part

    @pl.when(i > 0)
    def _():
        sums_ref[...] += part


def _post(acc, den, gf, Expand):
    return pl.pallas_call(
        _post_body,
        grid=(NG,),
        in_specs=[
            pl.BlockSpec((NC, NROW, MW), lambda i: (0, i, 0)),
            pl.BlockSpec((NW, NROW, H), lambda i: (0, i, 0)),
            pl.BlockSpec((NROW, 1), lambda i: (i, 0)),
            pl.BlockSpec((H, H * DH), lambda i: (0, 0)),
        ],
        out_specs=pl.BlockSpec((B, H * DH + 1), lambda i: (0, 0)),
        out_shape=jax.ShapeDtypeStruct((B, H * DH + 1), jnp.float32),
    )(acc, den, gf, Expand)


def _final_body(sums_ref, sc_ref, w2_ref, b2_ref, w3_ref, b3_ref, out_ref):
    sums = sums_ref[...]
    cnt = sums[:, H * DH:H * DH + 1]
    pooled = sums[:, :H * DH] / jnp.maximum(cnt, 1.0)
    proj = jnp.dot(pooled, w2_ref[...], preferred_element_type=jnp.float32)
    proj = jnp.maximum(proj + b2_ref[...], 0.0)
    feat = jnp.concatenate([proj, sc_ref[...]], axis=1)
    out_ref[...] = jnp.dot(feat, w3_ref[...],
                           preferred_element_type=jnp.float32) + b3_ref[...]


def _final(sums, scores, W2, b2, W3, b3):
    return pl.pallas_call(
        _final_body,
        out_shape=jax.ShapeDtypeStruct((B, 1), jnp.float32),
    )(sums, scores, W2, b2.reshape(1, PROJ), W3, b3.reshape(1, 1))


def kernel(x, edge_index, graph_ids, scores, W, a_src, a_dst, W2, b2, W3, b3):
    src = edge_index[0].astype(jnp.int32).reshape(NW, NBLK, KE)
    dst = edge_index[1].astype(jnp.int32).reshape(NW, NBLK, KE)

    rows = jnp.arange(D)
    head = rows // DH
    A2 = jnp.zeros((D, 2 * H), jnp.float32)
    A2 = A2.at[rows, head].set(a_src.reshape(-1))
    A2 = A2.at[rows, H + head].set(a_dst.reshape(-1))

    cols = jnp.arange(H * DH)
    Expand = (cols[None, :] // DH == jnp.arange(H)[:, None]).astype(jnp.float32)

    h, ee = _dense(x, W, A2)
    srcP = jnp.pad(src.reshape(NW, EPT), ((0, 0), (0, EPP - EPT)))
    dstP = jnp.pad(dst.reshape(NW, EPT), ((0, 0), (0, EPP - EPT)),
                   constant_values=N)
    eeP = jnp.pad(ee.reshape(N * 2 * H), (0, EEP - N * 2 * H))
    w, den = _wden(srcP, dstP, eeP)
    acc = _scat(src, dst, w, h)
    gf = graph_ids.astype(jnp.float32).reshape(N, 1)
    sums = _post(acc, den[:, :N * H].reshape(NW, N, H), gf, Expand)
    return _final(sums, scores, W2, b2, W3, b3)
